# baseline (speedup 1.0000x reference)
import functools

import jax
import jax.numpy as jnp
import numpy as np
from jax import lax
from jax.experimental import pallas as pl
from jax.experimental.pallas import tpu as pltpu
from jax.experimental.pallas import tpu_sc as plsc

_B, _L, _D = 16, 4096, 1024
_LANES = 16
_R = 4
_NCHUNKS = _L // _R
_SENT = _L


def _atext_body(seq_hbm, out_hbm, buf_v):
    sid = lax.axis_index("s")
    wid = sid

    @pl.when(wid < _B)
    def _():
        b = wid

        def row_max(r):
            acc = buf_v[r, pl.ds(0, _LANES)]
            for j in range(1, 8):
                acc = jnp.maximum(acc, buf_v[r, pl.ds(j * _LANES, _LANES)])

            def mbody(i, a):
                base = i * (8 * _LANES)
                for j in range(8):
                    a = jnp.maximum(a, buf_v[r, pl.ds(base + j * _LANES, _LANES)])
                return a

            acc = lax.fori_loop(1, _D // (8 * _LANES), mbody, acc)
            return jnp.max(acc)

        def chunk_cond(state):
            ci, fpos, _ = state
            return (fpos == _SENT) & (ci < _NCHUNKS)

        def chunk_body(state):
            ci, fpos, fzero = state
            pltpu.sync_copy(seq_hbm.at[b, pl.ds(ci * _R, _R)], buf_v)

            def row_cond(st):
                r, fp, _ = st
                return (fp == _SENT) & (r < _R)

            def row_body(st):
                r, fp, fz = st
                m = row_max(r)
                lidx = ci * _R + r
                fp = jnp.where(m > 0.0, lidx, fp)
                fz = jnp.where((fz == _SENT) & (m == 0.0), lidx, fz)
                return (r + np.int32(1), fp, fz)

            _, fpos, fzero = lax.while_loop(
                row_cond, row_body, (np.int32(0), fpos, fzero)
            )
            return (ci + np.int32(1), fpos, fzero)

        _, fpos, fzero = lax.while_loop(
            chunk_cond, chunk_body, (np.int32(0), np.int32(_SENT), np.int32(_SENT))
        )
        ans = jnp.where(
            fpos != _SENT, fpos, jnp.where(fzero != _SENT, fzero, np.int32(0))
        )
        pltpu.sync_copy(seq_hbm.at[b, ans], out_hbm.at[b])


@jax.jit
def _atext(sequence):
    mesh = plsc.VectorSubcoreMesh(
        core_axis_name="c", subcore_axis_name="s", num_cores=1
    )
    return pl.kernel(
        _atext_body,
        out_type=jax.ShapeDtypeStruct((_B, _D), jnp.float32),
        mesh=mesh,
        compiler_params=pltpu.CompilerParams(
            needs_layout_passes=False, skip_device_barrier=True
        ),
        scratch_types=[
            pltpu.VMEM((_R, _D), jnp.float32),
        ],
    )(sequence)


def kernel(sequence):
    return _atext(sequence)


# speedup vs baseline: 1.1274x; 1.1274x over previous
import functools

import jax
import jax.numpy as jnp
import numpy as np
from jax import lax
from jax.experimental import pallas as pl
from jax.experimental.pallas import tpu as pltpu
from jax.experimental.pallas import tpu_sc as plsc

_B, _L, _D = 16, 4096, 1024
_LANES = 16
_R = 4
_NCHUNKS = _L // _R
_SENT = _L


def _atext_body(seq_hbm, out_hbm, buf_v, row_v):
    sid = lax.axis_index("s")
    wid = sid

    @pl.when(wid < _B)
    def _():
        b = wid

        def row_max(r):
            acc = buf_v[r, pl.ds(0, _LANES)]
            for j in range(1, 8):
                acc = jnp.maximum(acc, buf_v[r, pl.ds(j * _LANES, _LANES)])

            def mbody(i, a):
                base = i * (8 * _LANES)
                for j in range(8):
                    a = jnp.maximum(a, buf_v[r, pl.ds(base + j * _LANES, _LANES)])
                return a

            acc = lax.fori_loop(1, _D // (8 * _LANES), mbody, acc)
            return jnp.max(acc)

        def chunk_cond(state):
            ci, fpos, _ = state
            return (fpos == _SENT) & (ci < _NCHUNKS)

        def chunk_body(state):
            ci, fpos, fzero = state
            pltpu.sync_copy(seq_hbm.at[b, pl.ds(ci * _R, _R)], buf_v)

            def row_cond(st):
                r, fp, _ = st
                return (fp == _SENT) & (r < _R)

            def row_body(st):
                r, fp, fz = st
                m = row_max(r)
                lidx = ci * _R + r
                fp = jnp.where(m > 0.0, lidx, fp)
                fz = jnp.where((fz == _SENT) & (m == 0.0), lidx, fz)
                return (r + np.int32(1), fp, fz)

            _, fpos, fzero = lax.while_loop(
                row_cond, row_body, (np.int32(0), fpos, fzero)
            )
            return (ci + np.int32(1), fpos, fzero)

        ci, fpos, fzero = lax.while_loop(
            chunk_cond, chunk_body, (np.int32(0), np.int32(_SENT), np.int32(_SENT))
        )
        ans = jnp.where(
            fpos != _SENT, fpos, jnp.where(fzero != _SENT, fzero, np.int32(0))
        )
        base = (ci - 1) * _R

        def from_buf(_):
            pltpu.sync_copy(buf_v.at[ans - base], out_hbm.at[b])
            return 0

        def from_hbm(_):
            pltpu.sync_copy(seq_hbm.at[b, ans], row_v)
            pltpu.sync_copy(row_v, out_hbm.at[b])
            return 0

        lax.cond(ans >= base, from_buf, from_hbm, 0)


@jax.jit
def _atext(sequence):
    mesh = plsc.VectorSubcoreMesh(
        core_axis_name="c", subcore_axis_name="s", num_cores=1
    )
    return pl.kernel(
        _atext_body,
        out_type=jax.ShapeDtypeStruct((_B, _D), jnp.float32),
        mesh=mesh,
        compiler_params=pltpu.CompilerParams(
            needs_layout_passes=False, skip_device_barrier=True
        ),
        scratch_types=[
            pltpu.VMEM((_R, _D), jnp.float32),
            pltpu.VMEM((_D,), jnp.float32),
        ],
    )(sequence)


def kernel(sequence):
    return _atext(sequence)


# speedup vs baseline: 1.1416x; 1.0126x over previous
import functools

import jax
import jax.numpy as jnp
import numpy as np
from jax import lax
from jax.experimental import pallas as pl
from jax.experimental.pallas import tpu as pltpu
from jax.experimental.pallas import tpu_sc as plsc

_B, _L, _D = 16, 4096, 1024
_LANES = 16
_R = 1
_NCHUNKS = _L // _R
_SENT = _L


def _atext_body(seq_hbm, out_hbm, buf_v, row_v):
    sid = lax.axis_index("s")
    wid = sid

    @pl.when(wid < _B)
    def _():
        b = wid

        def row_max(r):
            acc = buf_v[r, pl.ds(0, _LANES)]
            for j in range(1, 8):
                acc = jnp.maximum(acc, buf_v[r, pl.ds(j * _LANES, _LANES)])

            def mbody(i, a):
                base = i * (8 * _LANES)
                for j in range(8):
                    a = jnp.maximum(a, buf_v[r, pl.ds(base + j * _LANES, _LANES)])
                return a

            acc = lax.fori_loop(1, _D // (8 * _LANES), mbody, acc)
            return jnp.max(acc)

        def chunk_cond(state):
            ci, fpos, _ = state
            return (fpos == _SENT) & (ci < _NCHUNKS)

        def chunk_body(state):
            ci, fpos, fzero = state
            pltpu.sync_copy(seq_hbm.at[b, pl.ds(ci * _R, _R)], buf_v)
            m = row_max(0)
            fpos = jnp.where(m > 0.0, ci, fpos)
            fzero = jnp.where((fzero == _SENT) & (m == 0.0), ci, fzero)
            return (ci + np.int32(1), fpos, fzero)

        ci, fpos, fzero = lax.while_loop(
            chunk_cond, chunk_body, (np.int32(0), np.int32(_SENT), np.int32(_SENT))
        )
        ans = jnp.where(
            fpos != _SENT, fpos, jnp.where(fzero != _SENT, fzero, np.int32(0))
        )
        base = (ci - 1) * _R

        def from_buf(_):
            pltpu.sync_copy(buf_v.at[ans - base], out_hbm.at[b])
            return 0

        def from_hbm(_):
            pltpu.sync_copy(seq_hbm.at[b, ans], row_v)
            pltpu.sync_copy(row_v, out_hbm.at[b])
            return 0

        lax.cond(ans >= base, from_buf, from_hbm, 0)


@jax.jit
def _atext(sequence):
    mesh = plsc.VectorSubcoreMesh(
        core_axis_name="c", subcore_axis_name="s", num_cores=1
    )
    return pl.kernel(
        _atext_body,
        out_type=jax.ShapeDtypeStruct((_B, _D), jnp.float32),
        mesh=mesh,
        compiler_params=pltpu.CompilerParams(
            needs_layout_passes=False, skip_device_barrier=True
        ),
        scratch_types=[
            pltpu.VMEM((_R, _D), jnp.float32),
            pltpu.VMEM((_D,), jnp.float32),
        ],
    )(sequence)


def kernel(sequence):
    return _atext(sequence)
